# baseline (speedup 1.0000x reference)
import jax
import jax.numpy as jnp
from jax.experimental import pallas as pl
from jax.experimental.pallas import tpu as pltpu
from jax.experimental.pallas import tpu_sc as plsc

_B = 4
_T = 2048
_D = 4096
_STRIDE = 2.0
_WIDTH = 30.0
_T_SEC = _T * _STRIDE

_TS = 256
_T_TC = _T - _TS
_TT = 128
_SC_ROWS = 8


def _tc_body(x_ref, q_ref, out_ref):
    xb = x_ref[...].reshape(_B * _TT, _D).astype(jnp.bfloat16)
    qb = q_ref[...].astype(jnp.bfloat16)
    m4 = jax.lax.dot_general(qb, xb, (((1,), (1,)), ((), ())),
                             preferred_element_type=jnp.float32)
    m4r = m4.reshape(_B, _B, _TT)
    bi = jax.lax.broadcasted_iota(jnp.int32, (_B, _B, _TT), 0)
    gi = jax.lax.broadcasted_iota(jnp.int32, (_B, _B, _TT), 1)
    out_ref[...] = jnp.sum(m4r * (bi == gi).astype(jnp.float32), axis=1)


def _round_bf16(x):
    u = jax.lax.bitcast_convert_type(x, jnp.int32)
    lsb = jax.lax.shift_right_logical(u, 16) & 1
    r = (u + (32767 + lsb)) & jnp.int32(-65536)
    return jax.lax.bitcast_convert_type(r, jnp.float32)


def _sc_block_body(x_vmem, q_vmem, o_vmem):
    for r in range(_SC_ROWS):
        def step(i, accs):
            a0, a1, a2, a3 = accs
            outs = []
            for j, a in enumerate((a0, a1, a2, a3)):
                off = i * 64 + j * 16
                xc = _round_bf16(x_vmem[0, r, pl.ds(off, 16)])
                qc = q_vmem[0, pl.ds(off, 16)]
                outs.append(a + xc * qc)
            return tuple(outs)

        z = jnp.zeros((16,), jnp.float32)
        a0, a1, a2, a3 = jax.lax.fori_loop(0, _D // 64, step, (z, z, z, z))
        o_vmem[0, r, :] = (a0 + a1) + (a2 + a3)


def _sc_body(x_hbm, q_hbm, o_hbm):
    pltpu.emit_pipeline(
        _sc_block_body,
        grid=(_B, _TS // _SC_ROWS),
        in_specs=[
            pl.BlockSpec((1, _SC_ROWS, _D),
                         lambda b, t: (b, _T_TC // _SC_ROWS + t, 0)),
            pl.BlockSpec((1, _D), lambda b, t: (b, 0)),
        ],
        out_specs=[pl.BlockSpec((1, _SC_ROWS, 16), lambda b, t: (b, t, 0))],
        core_axis_name=("core", "subcore"),
        dimension_semantics=(pltpu.PARALLEL, pltpu.PARALLEL),
    )(x_hbm, q_hbm, o_hbm)


def _post_body(tc_ref, sc_ref, seg_ref, loss_ref, logits_ref, segs_ref,
               scores_ref):
    sc_logits = jnp.sum(sc_ref[...], axis=2)
    x = jnp.concatenate([tc_ref[...], sc_logits], axis=1)
    logits_ref[...] = x
    seg = jnp.round(seg_ref[...])
    tf = jax.lax.broadcasted_iota(jnp.int32, (_B, _T), 1).astype(jnp.float32)
    gt = ((tf >= seg[:, 0:1]) & (tf < seg[:, 1:2])).astype(jnp.float32)

    p = jax.nn.sigmoid(x)
    ce = jnp.maximum(x, 0.0) - x * gt + jnp.log1p(jnp.exp(-jnp.abs(x)))
    p_t = p * gt + (1.0 - p) * (1.0 - gt)
    foc = ce * (1.0 - p_t) ** 2
    a_t = 0.25 * gt + 0.75 * (1.0 - gt)
    loss_ref[...] = jnp.mean(a_t * foc).reshape(1, 1)

    ti = jax.lax.broadcasted_iota(jnp.int32, (_B, _T), 1)
    radius = int(_WIDTH / _STRIDE)
    scores = x
    ms, cs = [], []
    for _ in range(5):
        m = jnp.max(scores, axis=1, keepdims=True)
        i = jnp.min(jnp.where(scores == m, ti, _T), axis=1, keepdims=True)
        ms.append(m)
        cs.append(i.astype(jnp.float32) * _STRIDE)
        scores = jnp.where(jnp.abs(ti - i) < radius, -jnp.inf, scores)
    c = jnp.concatenate(cs, axis=1)
    scores_ref[...] = jnp.concatenate(ms, axis=1)
    lo = jnp.clip(c - _WIDTH / 2.0, 0.0, _T_SEC)
    hi = jnp.clip(c + _WIDTH / 2.0, 0.0, _T_SEC)
    segs_ref[...] = jnp.stack([lo, hi], axis=-1)


def kernel(z_ctx, m_ctx, z_q, gt_segment):
    del m_ctx

    q_rounded = _round_bf16(z_q)
    sc_kern = pl.kernel(
        _sc_body,
        out_type=jax.ShapeDtypeStruct((_B, _TS, 16), jnp.float32),
        mesh=plsc.VectorSubcoreMesh(core_axis_name="core",
                                    subcore_axis_name="subcore"),
        compiler_params=pltpu.CompilerParams(skip_device_barrier=True,
                                             has_side_effects=False),
    )
    logits_sc = sc_kern(z_ctx, q_rounded)

    logits_tc = pl.pallas_call(
        _tc_body,
        grid=(_T_TC // _TT,),
        in_specs=[
            pl.BlockSpec((_B, _TT, _D), lambda t: (0, t, 0)),
            pl.BlockSpec((_B, _D), lambda t: (0, 0)),
        ],
        out_specs=pl.BlockSpec((_B, _TT), lambda t: (0, t)),
        out_shape=jax.ShapeDtypeStruct((_B, _T_TC), jnp.float32),
        compiler_params=pltpu.CompilerParams(has_side_effects=False),
    )(z_ctx, z_q)

    loss, logits, segments, scores = pl.pallas_call(
        _post_body,
        out_shape=(
            jax.ShapeDtypeStruct((1, 1), jnp.float32),
            jax.ShapeDtypeStruct((_B, _T), jnp.float32),
            jax.ShapeDtypeStruct((_B, 5, 2), jnp.float32),
            jax.ShapeDtypeStruct((_B, 5), jnp.float32),
        ),
    )(logits_tc, logits_sc, gt_segment)

    return (loss[0, 0], logits, segments, scores)


# speedup vs baseline: 1.4782x; 1.4782x over previous
import jax
import jax.numpy as jnp
from jax.experimental import pallas as pl

_B = 4
_T = 2048
_D = 4096
_STRIDE = 2.0
_WIDTH = 30.0
_T_SEC = _T * _STRIDE
_TT = 128


def _body(x_ref, q_ref, seg_ref, loss_ref, logits_ref, segs_ref, scores_ref):
    t = pl.program_id(0)
    xb = x_ref[...].reshape(_B * _TT, _D).astype(jnp.bfloat16)
    qb = q_ref[...].astype(jnp.bfloat16)
    m4 = jax.lax.dot_general(qb, xb, (((1,), (1,)), ((), ())),
                             preferred_element_type=jnp.float32)
    m4r = m4.reshape(_B, _B, _TT)
    bi = jax.lax.broadcasted_iota(jnp.int32, (_B, _B, _TT), 0)
    gi = jax.lax.broadcasted_iota(jnp.int32, (_B, _B, _TT), 1)
    blk = jnp.sum(m4r * (bi == gi).astype(jnp.float32), axis=1)
    logits_ref[:, pl.ds(t * _TT, _TT)] = blk

    @pl.when(t == _T // _TT - 1)
    def _epilogue():
        x = logits_ref[...]
        seg = jnp.round(seg_ref[...])
        tf = jax.lax.broadcasted_iota(jnp.int32, (_B, _T), 1).astype(jnp.float32)
        gt = ((tf >= seg[:, 0:1]) & (tf < seg[:, 1:2])).astype(jnp.float32)

        p = jax.nn.sigmoid(x)
        ce = jnp.maximum(x, 0.0) - x * gt + jnp.log1p(jnp.exp(-jnp.abs(x)))
        p_t = p * gt + (1.0 - p) * (1.0 - gt)
        foc = ce * (1.0 - p_t) ** 2
        a_t = 0.25 * gt + 0.75 * (1.0 - gt)
        loss_ref[...] = jnp.mean(a_t * foc).reshape(1, 1)

        ti = jax.lax.broadcasted_iota(jnp.int32, (_B, _T), 1)
        radius = int(_WIDTH / _STRIDE)
        scores = x
        ms, cs = [], []
        for _ in range(5):
            m = jnp.max(scores, axis=1, keepdims=True)
            i = jnp.min(jnp.where(scores == m, ti, _T), axis=1, keepdims=True)
            ms.append(m)
            cs.append(i.astype(jnp.float32) * _STRIDE)
            scores = jnp.where(jnp.abs(ti - i) < radius, -jnp.inf, scores)
        c = jnp.concatenate(cs, axis=1)
        scores_ref[...] = jnp.concatenate(ms, axis=1)
        lo = jnp.clip(c - _WIDTH / 2.0, 0.0, _T_SEC)
        hi = jnp.clip(c + _WIDTH / 2.0, 0.0, _T_SEC)
        segs_ref[...] = jnp.stack([lo, hi], axis=-1)


def kernel(z_ctx, m_ctx, z_q, gt_segment):
    del m_ctx
    loss, logits, segments, scores = pl.pallas_call(
        _body,
        grid=(_T // _TT,),
        in_specs=[
            pl.BlockSpec((_B, _TT, _D), lambda t: (0, t, 0)),
            pl.BlockSpec((_B, _D), lambda t: (0, 0)),
            pl.BlockSpec((_B, 2), lambda t: (0, 0)),
        ],
        out_specs=(
            pl.BlockSpec((1, 1), lambda t: (0, 0)),
            pl.BlockSpec((_B, _T), lambda t: (0, 0)),
            pl.BlockSpec((_B, 5, 2), lambda t: (0, 0, 0)),
            pl.BlockSpec((_B, 5), lambda t: (0, 0)),
        ),
        out_shape=(
            jax.ShapeDtypeStruct((1, 1), jnp.float32),
            jax.ShapeDtypeStruct((_B, _T), jnp.float32),
            jax.ShapeDtypeStruct((_B, 5, 2), jnp.float32),
            jax.ShapeDtypeStruct((_B, 5), jnp.float32),
        ),
    )(z_ctx, z_q, gt_segment)

    return (loss[0, 0], logits, segments, scores)
